# baseline (speedup 1.0000x reference)
import dataclasses
import functools

import jax
import jax.numpy as jnp
from jax import lax
from jax.experimental import pallas as pl
from jax.experimental.pallas import tpu as pltpu
from jax.experimental.pallas import tpu_sc as plsc

N = 4096
E = 16384
D = 1024
BLK = 256
NC = 2
NS = 16
NW = NC * NS
EA = E // NW
COLS_PER_TILE = D // NW
SLAB = 8
NPASS = COLS_PER_TILE // SLAB
EW = 2048

_MESH = plsc.VectorSubcoreMesh(core_axis_name="c", subcore_axis_name="s")
_HIGHEST = lax.Precision.HIGHEST

_SC_PARAMS = pltpu.CompilerParams()
if "needs_layout_passes" in pltpu.CompilerParams.__dataclass_fields__:
    _SC_PARAMS = dataclasses.replace(_SC_PARAMS, needs_layout_passes=False)


def _mm_body(x_ref, w_ref, att_ref, ht_ref, sl_ref, sr_ref,
             xh_ref, xl_ref, wh_ref, wl_ref):
    i = pl.program_id(0)
    j = pl.program_id(1)

    @pl.when(jnp.logical_and(i == 0, j == 0))
    def _():
        wf = w_ref[...]
        wh = wf.astype(jnp.bfloat16)
        wh_ref[...] = wh
        wl_ref[...] = (wf - wh.astype(jnp.float32)).astype(jnp.bfloat16)

    @pl.when(j == 0)
    def _():
        xf = x_ref[...]
        xh = xf.astype(jnp.bfloat16)
        xh_ref[...] = xh
        xl_ref[...] = (xf - xh.astype(jnp.float32)).astype(jnp.bfloat16)

    xh = xh_ref[...]
    xl = xl_ref[...]
    wh = wh_ref[pl.ds(j * BLK, BLK), :]
    wl = wl_ref[pl.ds(j * BLK, BLK), :]
    dn = (((1,), (1,)), ((), ()))
    ht = (lax.dot_general(wh, xh, dn, preferred_element_type=jnp.float32)
          + lax.dot_general(wh, xl, dn, preferred_element_type=jnp.float32)
          + lax.dot_general(wl, xh, dn, preferred_element_type=jnp.float32))
    ht_ref[...] = ht
    al = att_ref[0, pl.ds(j * BLK, BLK)]
    ar = att_ref[0, pl.ds(D + j * BLK, BLK)]
    psl = jnp.sum(ht * al[:, None], axis=0)
    psr = jnp.sum(ht * ar[:, None], axis=0)

    @pl.when(j == 0)
    def _():
        sl_ref[...] = psl
        sr_ref[...] = psr

    @pl.when(j != 0)
    def _():
        sl_ref[...] += psl
        sr_ref[...] += psr


_mm_call = pl.pallas_call(
    _mm_body,
    grid=(N // BLK, D // BLK),
    in_specs=[
        pl.BlockSpec((BLK, D), lambda i, j: (i, 0)),
        pl.BlockSpec((D, D), lambda i, j: (0, 0)),
        pl.BlockSpec((1, 2 * D), lambda i, j: (0, 0)),
    ],
    out_specs=[
        pl.BlockSpec((BLK, BLK), lambda i, j: (j, i)),
        pl.BlockSpec((BLK,), lambda i, j: (i,)),
        pl.BlockSpec((BLK,), lambda i, j: (i,)),
    ],
    out_shape=[
        jax.ShapeDtypeStruct((D, N), jnp.float32),
        jax.ShapeDtypeStruct((N,), jnp.float32),
        jax.ShapeDtypeStruct((N,), jnp.float32),
    ],
    scratch_shapes=[
        pltpu.VMEM((BLK, D), jnp.bfloat16),
        pltpu.VMEM((BLK, D), jnp.bfloat16),
        pltpu.VMEM((D, D), jnp.bfloat16),
        pltpu.VMEM((D, D), jnp.bfloat16),
    ],
)


def _agg_body(ht_hbm, sl_hbm, sr_hbm, src_hbm, dst_hbm, agg_hbm, dp_hbm,
              slab, agg, src_v, dst_v, p_v, sl_v, sr_v, part, sem, sem2):
    c = lax.axis_index("c")
    s = lax.axis_index("s")
    w = c * NS + s
    col0 = w * COLS_PER_TILE

    z16 = jnp.zeros((16,), jnp.float32)
    jfull = [jnp.full((16,), j, jnp.int32) for j in range(SLAB)]

    cp = pltpu.async_copy(ht_hbm.at[pl.ds(col0, SLAB)], slab, sem2)
    pltpu.async_copy(sl_hbm, sl_v, sem).wait()
    pltpu.async_copy(sr_hbm, sr_v, sem).wait()
    pltpu.async_copy(src_hbm, src_v, sem).wait()
    pltpu.async_copy(dst_hbm, dst_v, sem).wait()

    @plsc.parallel_loop(0, E // 16, 1, unroll=4)
    def _(g):
        sv = src_v[pl.ds(g * 16, 16)]
        dv = dst_v[pl.ds(g * 16, 16)]
        a = plsc.load_gather(sl_v, [sv]) + plsc.load_gather(sr_v, [dv])
        alpha = jnp.where(a > 0, a, 0.2 * a)
        p_v[pl.ds(g * 16, 16)] = jnp.exp(alpha)

    @pl.loop(0, N // 16)
    def _(i):
        part[pl.ds(i * 16, 16)] = z16

    e0 = w * EA
    lane0 = lax.iota(jnp.int32, 16) == 0

    @pl.loop(0, EA)
    def _(i):
        ev = jnp.full((16,), e0 + i, jnp.int32)
        dsp = plsc.load_gather(dst_v, [ev])
        qv = plsc.load_gather(p_v, [ev]) - 1.0
        plsc.addupdate_scatter(part, [dsp], qv, mask=lane0)

    pltpu.sync_copy(part, dp_hbm.at[w])

    for t in range(NPASS):
        @pl.loop(0, N // 16)
        def _(i):
            for j in range(SLAB):
                agg[j, pl.ds(i * 16, 16)] = z16

        cp.wait()

        @plsc.parallel_loop(0, E // 16, 1, unroll=6)
        def _(g):
            sv = src_v[pl.ds(g * 16, 16)]
            dv = dst_v[pl.ds(g * 16, 16)]
            pv = p_v[pl.ds(g * 16, 16)]
            for j in range(SLAB):
                vals = plsc.load_gather(slab, [jfull[j], sv]) * pv
                plsc.addupdate_scatter(agg, [jfull[j], dv], vals)

        if t + 1 < NPASS:
            cp = pltpu.async_copy(
                ht_hbm.at[pl.ds(col0 + (t + 1) * SLAB, SLAB)], slab, sem2)
        pltpu.sync_copy(agg, agg_hbm.at[pl.ds(col0 + t * SLAB, SLAB)])


_agg_call = functools.partial(
    pl.kernel,
    out_type=[
        jax.ShapeDtypeStruct((D, N), jnp.float32),
        jax.ShapeDtypeStruct((NW, N), jnp.float32),
    ],
    mesh=_MESH,
    compiler_params=_SC_PARAMS,
    scratch_types=[
        pltpu.VMEM((SLAB, N), jnp.float32),
        pltpu.VMEM((SLAB, N), jnp.float32),
        pltpu.VMEM((E,), jnp.int32),
        pltpu.VMEM((E,), jnp.int32),
        pltpu.VMEM((E,), jnp.float32),
        pltpu.VMEM((N,), jnp.float32),
        pltpu.VMEM((N,), jnp.float32),
        pltpu.VMEM((N,), jnp.float32),
        pltpu.SemaphoreType.DMA,
        pltpu.SemaphoreType.DMA,
    ],
)(_agg_body)


def _fin_body(aggt_ref, dp_ref, out_ref):
    den = float(E) + jnp.sum(dp_ref[...], axis=0)
    a = aggt_ref[...].T / den[:, None]
    out_ref[...] = jnp.where(a > 0, a, jnp.exp(a) - 1.0)


_fin_call = pl.pallas_call(
    _fin_body,
    grid=(N // BLK, D // BLK),
    in_specs=[
        pl.BlockSpec((BLK, BLK), lambda i, j: (j, i)),
        pl.BlockSpec((NW, BLK), lambda i, j: (0, i)),
    ],
    out_specs=pl.BlockSpec((BLK, BLK), lambda i, j: (i, j)),
    out_shape=jax.ShapeDtypeStruct((N, D), jnp.float32),
)


def kernel(x, edge_index, W, att):
    src = edge_index[0]
    dst = edge_index[1]
    hT, s_l, s_r = _mm_call(x, W, att)
    aggT, dp = _agg_call(hT, s_l, s_r, src, dst)
    return _fin_call(aggT, dp)


# speedup vs baseline: 1.1219x; 1.1219x over previous
import dataclasses
import functools

import jax
import jax.numpy as jnp
from jax import lax
from jax.experimental import pallas as pl
from jax.experimental.pallas import tpu as pltpu
from jax.experimental.pallas import tpu_sc as plsc

N = 4096
E = 16384
D = 1024
BLK = 256
NC = 2
NS = 16
NW = NC * NS
EA = E // NW
COLS_PER_TILE = D // NW
SLAB = 8
NPASS = COLS_PER_TILE // SLAB
EW = 2048

_MESH = plsc.VectorSubcoreMesh(core_axis_name="c", subcore_axis_name="s")

_SC_PARAMS = pltpu.CompilerParams()
if "needs_layout_passes" in pltpu.CompilerParams.__dataclass_fields__:
    _SC_PARAMS = dataclasses.replace(_SC_PARAMS, needs_layout_passes=False)


def _mm_body(x_ref, w_ref, att_ref, ht_ref, sl_ref, sr_ref,
             xh_ref, xl_ref, wh_ref, wl_ref):
    i = pl.program_id(0)
    j = pl.program_id(1)

    @pl.when(jnp.logical_and(i == 0, j == 0))
    def _():
        wf = w_ref[...]
        wh = wf.astype(jnp.bfloat16)
        wh_ref[...] = wh
        wl_ref[...] = (wf - wh.astype(jnp.float32)).astype(jnp.bfloat16)

    @pl.when(j == 0)
    def _():
        xf = x_ref[...]
        xh = xf.astype(jnp.bfloat16)
        xh_ref[...] = xh
        xl_ref[...] = (xf - xh.astype(jnp.float32)).astype(jnp.bfloat16)

    xh = xh_ref[...]
    xl = xl_ref[...]
    wh = wh_ref[pl.ds(j * BLK, BLK), :]
    wl = wl_ref[pl.ds(j * BLK, BLK), :]
    dn = (((1,), (1,)), ((), ()))
    ht = (lax.dot_general(wh, xh, dn, preferred_element_type=jnp.float32)
          + lax.dot_general(wh, xl, dn, preferred_element_type=jnp.float32)
          + lax.dot_general(wl, xh, dn, preferred_element_type=jnp.float32))
    ht_ref[...] = ht
    al = att_ref[0, pl.ds(j * BLK, BLK)]
    ar = att_ref[0, pl.ds(D + j * BLK, BLK)]
    psl = jnp.sum(ht * al[:, None], axis=0)
    psr = jnp.sum(ht * ar[:, None], axis=0)

    @pl.when(j == 0)
    def _():
        sl_ref[...] = psl
        sr_ref[...] = psr

    @pl.when(j != 0)
    def _():
        sl_ref[...] += psl
        sr_ref[...] += psr


_mm_call = pl.pallas_call(
    _mm_body,
    grid=(N // BLK, D // BLK),
    in_specs=[
        pl.BlockSpec((BLK, D), lambda i, j: (i, 0)),
        pl.BlockSpec((D, D), lambda i, j: (0, 0)),
        pl.BlockSpec((1, 2 * D), lambda i, j: (0, 0)),
    ],
    out_specs=[
        pl.BlockSpec((BLK, BLK), lambda i, j: (j, i)),
        pl.BlockSpec((BLK,), lambda i, j: (i,)),
        pl.BlockSpec((BLK,), lambda i, j: (i,)),
    ],
    out_shape=[
        jax.ShapeDtypeStruct((D, N), jnp.float32),
        jax.ShapeDtypeStruct((N,), jnp.float32),
        jax.ShapeDtypeStruct((N,), jnp.float32),
    ],
    scratch_shapes=[
        pltpu.VMEM((BLK, D), jnp.bfloat16),
        pltpu.VMEM((BLK, D), jnp.bfloat16),
        pltpu.VMEM((D, D), jnp.bfloat16),
        pltpu.VMEM((D, D), jnp.bfloat16),
    ],
)


def _agg_body(ht_hbm, sl_hbm, sr_hbm, src_hbm, dst_hbm, agg_hbm, dp_hbm,
              slab, agg, src_v, dst_v, p_v, sl_v, sr_v, part, sem, sem2):
    c = lax.axis_index("c")
    s = lax.axis_index("s")
    w = c * NS + s
    col0 = w * COLS_PER_TILE

    z16 = jnp.zeros((16,), jnp.float32)
    jfull = [jnp.full((16,), j, jnp.int32) for j in range(SLAB)]

    cp = pltpu.async_copy(ht_hbm.at[pl.ds(col0, SLAB)], slab, sem2)
    pltpu.async_copy(sl_hbm, sl_v, sem).wait()
    pltpu.async_copy(sr_hbm, sr_v, sem).wait()
    pltpu.async_copy(src_hbm, src_v, sem).wait()
    pltpu.async_copy(dst_hbm, dst_v, sem).wait()

    @plsc.parallel_loop(0, E // 16, 1, unroll=4)
    def _(g):
        sv = src_v[pl.ds(g * 16, 16)]
        dv = dst_v[pl.ds(g * 16, 16)]
        a = plsc.load_gather(sl_v, [sv]) + plsc.load_gather(sr_v, [dv])
        alpha = jnp.where(a > 0, a, 0.2 * a)
        p_v[pl.ds(g * 16, 16)] = jnp.exp(alpha)

    @pl.loop(0, N // 16)
    def _(i):
        part[pl.ds(i * 16, 16)] = z16

    e0 = w * EA
    lane0 = lax.iota(jnp.int32, 16) == 0

    @pl.loop(0, EA)
    def _(i):
        ev = jnp.full((16,), e0 + i, jnp.int32)
        dsp = plsc.load_gather(dst_v, [ev])
        qv = plsc.load_gather(p_v, [ev]) - 1.0
        plsc.addupdate_scatter(part, [dsp], qv, mask=lane0)

    pltpu.sync_copy(part, dp_hbm.at[w])

    for t in range(NPASS):
        @pl.loop(0, N // 16)
        def _(i):
            for j in range(SLAB):
                agg[j, pl.ds(i * 16, 16)] = z16

        cp.wait()

        @plsc.parallel_loop(0, E // 16, 1, unroll=4)
        def _(g):
            sv = src_v[pl.ds(g * 16, 16)]
            dv = dst_v[pl.ds(g * 16, 16)]
            pv = p_v[pl.ds(g * 16, 16)]
            for j in range(SLAB):
                vals = plsc.load_gather(slab, [jfull[j], sv]) * pv
                plsc.addupdate_scatter(agg, [jfull[j], dv], vals)

        if t + 1 < NPASS:
            cp = pltpu.async_copy(
                ht_hbm.at[pl.ds(col0 + (t + 1) * SLAB, SLAB)], slab, sem2)
        pltpu.sync_copy(agg, agg_hbm.at[pl.ds(col0 + t * SLAB, SLAB)])


_agg_call = functools.partial(
    pl.kernel,
    out_type=[
        jax.ShapeDtypeStruct((D, N), jnp.float32),
        jax.ShapeDtypeStruct((NW, N), jnp.float32),
    ],
    mesh=_MESH,
    compiler_params=_SC_PARAMS,
    scratch_types=[
        pltpu.VMEM((SLAB, N), jnp.float32),
        pltpu.VMEM((SLAB, N), jnp.float32),
        pltpu.VMEM((E,), jnp.int32),
        pltpu.VMEM((E,), jnp.int32),
        pltpu.VMEM((E,), jnp.float32),
        pltpu.VMEM((N,), jnp.float32),
        pltpu.VMEM((N,), jnp.float32),
        pltpu.VMEM((N,), jnp.float32),
        pltpu.SemaphoreType.DMA,
        pltpu.SemaphoreType.DMA,
    ],
)(_agg_body)


def _fin_body(aggt_ref, dp_ref, out_ref):
    den = float(E) + jnp.sum(dp_ref[...], axis=0)
    a = aggt_ref[...].T / den[:, None]
    out_ref[...] = jnp.where(a > 0, a, jnp.exp(a) - 1.0)


_fin_call = pl.pallas_call(
    _fin_body,
    grid=(N // BLK, D // BLK),
    in_specs=[
        pl.BlockSpec((BLK, BLK), lambda i, j: (j, i)),
        pl.BlockSpec((NW, BLK), lambda i, j: (0, i)),
    ],
    out_specs=pl.BlockSpec((BLK, BLK), lambda i, j: (i, j)),
    out_shape=jax.ShapeDtypeStruct((N, D), jnp.float32),
)


def kernel(x, edge_index, W, att):
    src = edge_index[0]
    dst = edge_index[1]
    hT, s_l, s_r = _mm_call(x, W, att)
    aggT, dp = _agg_call(hT, s_l, s_r, src, dst)
    return _fin_call(aggT, dp)
